# baseline (speedup 1.0000x reference)
import functools

import jax
import jax.numpy as jnp
from jax import lax
from jax.experimental import pallas as pl
from jax.experimental.pallas import tpu as pltpu
from jax.experimental.pallas import tpu_sc as plsc

CLASSES = 100000
DIMS = 64
BATCH = 16384
EPS = 1e-09

NC, NS, L = 2, 16, 16
NW = NC * NS
BPW = BATCH // NW
IDX_CHUNKS = BPW // 128
UNROLL = 4


def _loss_body(pred_hbm, yt_hbm, pol_hbm, out_hbm,
               idx_v, rows_v, pred_v, stage_v, gsem, psem):
    wid = lax.axis_index("s") * NC + lax.axis_index("c")
    base = wid * BPW

    pltpu.sync_copy(yt_hbm.at[pl.ds(wid * IDX_CHUNKS, IDX_CHUNKS)], idx_v)

    pred_cp = pltpu.async_copy(pred_hbm.at[pl.ds(base, BPW)], pred_v, psem)
    gathers = [
        pltpu.async_copy(pol_hbm.at[idx_v.at[j]],
                         rows_v.at[pl.ds(j * 128, 128)], gsem)
        for j in range(IDX_CHUNKS)
    ]
    for g in gathers:
        g.wait()
    pred_cp.wait()

    half = jnp.float32(0.5)
    three_half = jnp.float32(1.5)
    one = jnp.float32(1.0)

    def loss_one(i):
        dot = None
        n1 = None
        n2 = None
        for k in range(DIMS // L):
            pv = pred_v[i, pl.ds(k * L, L)]
            tv = rows_v[i, pl.ds(k * L, L)]
            if k == 0:
                dot, n1, n2 = pv * tv, pv * pv, tv * tv
            else:
                dot = dot + pv * tv
                n1 = n1 + pv * pv
                n2 = n2 + tv * tv
        dots = jnp.sum(dot)
        n1s = jnp.sum(n1)
        n2s = jnp.sum(n2)
        prod = jnp.maximum(n1s * n2s, jnp.float32(1e-30))
        bits = lax.bitcast_convert_type(prod, jnp.int32)
        y = lax.bitcast_convert_type(
            jnp.int32(0x5F3759DF) - (bits >> 1), jnp.float32)
        for _ in range(3):
            y = y * (three_half - half * prod * y * y)
        scale = jnp.where(prod >= jnp.float32(EPS * EPS), y,
                          jnp.float32(1.0 / EPS))
        cos = dots * scale
        e = one - cos
        return e * e

    def main_body(it, acc):
        i0 = it * UNROLL
        for u in range(UNROLL):
            acc = acc + loss_one(i0 + u)
        return acc

    acc = lax.fori_loop(0, BPW // UNROLL, main_body, jnp.float32(0.0))
    stage_v[...] = jnp.full((L,), acc * jnp.float32(1.0 / (BATCH * L)))
    pltpu.sync_copy(stage_v, out_hbm.at[wid])


_sc_loss = functools.partial(
    pl.kernel,
    mesh=plsc.VectorSubcoreMesh(core_axis_name="c", subcore_axis_name="s"),
    out_type=jax.ShapeDtypeStruct((NW, L), jnp.float32),
    compiler_params=pltpu.CompilerParams(
        needs_layout_passes=False, use_tc_tiling_on_sc=False),
    scratch_types=[
        pltpu.VMEM((IDX_CHUNKS, 128), jnp.int32),
        pltpu.VMEM((BPW, DIMS), jnp.float32),
        pltpu.VMEM((BPW, DIMS), jnp.float32),
        pltpu.VMEM((L,), jnp.float32),
        pltpu.SemaphoreType.DMA,
        pltpu.SemaphoreType.DMA,
    ],
)(_loss_body)


def kernel(y_pred, y_true, polars):
    yt = y_true.astype(jnp.int32).reshape(BATCH // 128, 128)
    partials = _sc_loss(y_pred, yt, polars)
    return jnp.sum(partials)
